# Initial kernel scaffold; baseline (speedup 1.0000x reference)
import jax
import jax.numpy as jnp
from jax.experimental import pallas as pl


def kernel(z, emb_weight):
    raise NotImplementedError("write your pallas kernel here")


# speedup vs baseline: 1.1976x; 1.1976x over previous
import functools

import jax
import jax.numpy as jnp
from jax import lax
from jax.experimental import pallas as pl
from jax.experimental.pallas import tpu as pltpu
from jax.experimental.pallas import tpu_sc as plsc

TOK = 4608
D = 64
V = 8192
TBLK = 512
CBLK = 2048
NW = 32
BPW = TOK // NW
HALF = BPW // 2


def _argmin_body(z_ref, embt_ref, idx_ref, lsum_ref):
    i = pl.program_id(0)
    zb = z_ref[...]
    zsq = jnp.sum(zb * zb, axis=1, keepdims=True)
    best_d = None
    best_i = None
    for j in range(V // CBLK):
        eb = embt_ref[:, j * CBLK:(j + 1) * CBLK]
        esq = jnp.sum(eb * eb, axis=0, keepdims=True)
        c = lax.dot_general(zb, eb, (((1,), (0,)), ((), ())),
                            preferred_element_type=jnp.float32)
        d = (zsq + esq) - 2.0 * c
        dmin = jnp.min(d, axis=1, keepdims=True)
        ii = lax.broadcasted_iota(jnp.int32, d.shape, 1)
        imin = jnp.min(jnp.where(d == dmin, ii, V), axis=1,
                       keepdims=True) + j * CBLK
        if j == 0:
            best_d, best_i = dmin, imin
        else:
            upd = dmin < best_d
            best_i = jnp.where(upd, imin, best_i)
            best_d = jnp.where(upd, dmin, best_d)
    idx_ref[...] = best_i[:, 0]
    s = jnp.sum(best_d).reshape(1, 1)

    @pl.when(i == 0)
    def _():
        lsum_ref[...] = s

    @pl.when(i > 0)
    def _():
        lsum_ref[...] = lsum_ref[...] + s


def _argmin_call(z_flat, emb_t):
    return pl.pallas_call(
        _argmin_body,
        grid=(TOK // TBLK,),
        in_specs=[pl.BlockSpec((TBLK, D), lambda i: (i, 0)),
                  pl.BlockSpec((D, V), lambda i: (0, 0))],
        out_specs=[pl.BlockSpec((TBLK,), lambda i: (i,)),
                   pl.BlockSpec((1, 1), lambda i: (0, 0))],
        out_shape=[jax.ShapeDtypeStruct((TOK,), jnp.int32),
                   jax.ShapeDtypeStruct((1, 1), jnp.float32)],
    )(z_flat, emb_t)


DPAD = 128


@functools.lru_cache(maxsize=1)
def _make_gather_rows():
    @functools.partial(
        pl.kernel,
        mesh=plsc.VectorSubcoreMesh(core_axis_name="c", subcore_axis_name="s"),
        out_type=jax.ShapeDtypeStruct((TOK, DPAD), jnp.float32),
        scratch_types=[pltpu.VMEM((BPW,), jnp.int32),
                       pltpu.VMEM((BPW, DPAD), jnp.float32),
                       pltpu.SemaphoreType.DMA],
    )
    def _gather_rows(table_hbm, idx_hbm, out_hbm, idx_v, rows_v, sem):
        wid = lax.axis_index("s") * 2 + lax.axis_index("c")
        base = wid * BPW
        pltpu.sync_copy(idx_hbm.at[pl.ds(base, BPW)], idx_v)
        cp0 = pltpu.async_copy(table_hbm.at[idx_v.at[pl.ds(0, HALF)]],
                               rows_v.at[pl.ds(0, HALF)], sem)
        cp1 = pltpu.async_copy(table_hbm.at[idx_v.at[pl.ds(HALF, HALF)]],
                               rows_v.at[pl.ds(HALF, HALF)], sem)
        cp0.wait()
        cp1.wait()
        pltpu.sync_copy(rows_v, out_hbm.at[pl.ds(base, BPW)])

    return _gather_rows


def kernel(z, emb_weight):
    z = z.astype(jnp.float32)
    b, n, dim = z.shape
    z_flat = z.reshape(-1, dim)
    idx_flat, lsum = _argmin_call(z_flat, emb_weight.T)
    emb_pad = jnp.pad(emb_weight, ((0, 0), (0, DPAD - dim)))
    quantized = _make_gather_rows()(emb_pad, idx_flat)[:, :dim].reshape(z.shape)
    mse = lsum[0, 0] / (TOK * D)
    loss = 0.25 * mse + 1.0 * mse
    quantized_ste = z + (quantized - z)
    zero = jnp.zeros((), jnp.float32)
    return (z, emb_weight, quantized_ste, idx_flat.reshape(b, n), loss,
            mse, mse, zero, zero, zero)


# speedup vs baseline: 1.4489x; 1.2099x over previous
import functools

import jax
import jax.numpy as jnp
from jax import lax
from jax.experimental import pallas as pl
from jax.experimental.pallas import tpu as pltpu
from jax.experimental.pallas import tpu_sc as plsc

TOK = 4608
D = 64
V = 8192
TBLK = 512
CBLK = 2048
NW = 32
BPW = TOK // NW
HALF = BPW // 2


RG = 8
RROWS = TBLK // RG
KL = 128


def _argmin_body(z_ref, embt_ref, idx_ref, lsum_ref):
    i = pl.program_id(0)
    zb = z_ref[...]
    zsq = jnp.sum(zb * zb, axis=1, keepdims=True)
    zb2 = zb + zb
    m = [None] * RG
    g = [None] * RG
    for j in range(V // CBLK):
        eb = embt_ref[:, j * CBLK:(j + 1) * CBLK]
        esq = jnp.sum(eb * eb, axis=0, keepdims=True)
        c2 = lax.dot_general(zb2, eb, (((1,), (0,)), ((), ())),
                             preferred_element_type=jnp.float32)
        for rg in range(RG):
            zs = zsq[rg * RROWS:(rg + 1) * RROWS, :]
            for k in range(CBLK // KL):
                d = ((zs + esq[:, k * KL:(k + 1) * KL])
                     - c2[rg * RROWS:(rg + 1) * RROWS, k * KL:(k + 1) * KL])
                gid = j * (CBLK // KL) + k
                if gid == 0:
                    m[rg] = d
                    g[rg] = jnp.zeros(d.shape, jnp.float32)
                else:
                    upd = d < m[rg]
                    m[rg] = jnp.where(upd, d, m[rg])
                    g[rg] = jnp.where(upd, jnp.float32(gid), g[rg])
    lane = lax.broadcasted_iota(jnp.int32, (RROWS, KL), 1).astype(jnp.float32)
    idx_parts = []
    s = jnp.zeros((1, 1), jnp.float32)
    for rg in range(RG):
        dmin = jnp.min(m[rg], axis=1, keepdims=True)
        cval = g[rg] * 128.0 + lane
        sel = jnp.where(m[rg] == dmin, cval, jnp.float32(2 * V))
        imin = jnp.min(sel, axis=1)
        idx_parts.append(imin.astype(jnp.int32))
        s = s + jnp.sum(dmin).reshape(1, 1)
    idx_ref[...] = jnp.concatenate(idx_parts)

    @pl.when(i == 0)
    def _():
        lsum_ref[...] = s

    @pl.when(i > 0)
    def _():
        lsum_ref[...] = lsum_ref[...] + s


def _argmin_call(z_flat, emb_t):
    return pl.pallas_call(
        _argmin_body,
        grid=(TOK // TBLK,),
        in_specs=[pl.BlockSpec((TBLK, D), lambda i: (i, 0)),
                  pl.BlockSpec((D, V), lambda i: (0, 0))],
        out_specs=[pl.BlockSpec((TBLK,), lambda i: (i,)),
                   pl.BlockSpec((1, 1), lambda i: (0, 0))],
        out_shape=[jax.ShapeDtypeStruct((TOK,), jnp.int32),
                   jax.ShapeDtypeStruct((1, 1), jnp.float32)],
    )(z_flat, emb_t)


DPAD = 128


@functools.lru_cache(maxsize=1)
def _make_gather_rows():
    @functools.partial(
        pl.kernel,
        mesh=plsc.VectorSubcoreMesh(core_axis_name="c", subcore_axis_name="s"),
        out_type=jax.ShapeDtypeStruct((TOK, DPAD), jnp.float32),
        scratch_types=[pltpu.VMEM((BPW,), jnp.int32),
                       pltpu.VMEM((BPW, DPAD), jnp.float32),
                       pltpu.SemaphoreType.DMA],
    )
    def _gather_rows(table_hbm, idx_hbm, out_hbm, idx_v, rows_v, sem):
        wid = lax.axis_index("s") * 2 + lax.axis_index("c")
        base = wid * BPW
        pltpu.sync_copy(idx_hbm.at[pl.ds(base, BPW)], idx_v)
        cp0 = pltpu.async_copy(table_hbm.at[idx_v.at[pl.ds(0, HALF)]],
                               rows_v.at[pl.ds(0, HALF)], sem)
        cp1 = pltpu.async_copy(table_hbm.at[idx_v.at[pl.ds(HALF, HALF)]],
                               rows_v.at[pl.ds(HALF, HALF)], sem)
        cp0.wait()
        cp1.wait()
        pltpu.sync_copy(rows_v, out_hbm.at[pl.ds(base, BPW)])

    return _gather_rows


def kernel(z, emb_weight):
    z = z.astype(jnp.float32)
    b, n, dim = z.shape
    z_flat = z.reshape(-1, dim)
    idx_flat, lsum = _argmin_call(z_flat, emb_weight.T)
    emb_pad = jnp.pad(emb_weight, ((0, 0), (0, DPAD - dim)))
    quantized = _make_gather_rows()(emb_pad, idx_flat)[:, :dim].reshape(z.shape)
    mse = lsum[0, 0] / (TOK * D)
    loss = 0.25 * mse + 1.0 * mse
    quantized_ste = z + (quantized - z)
    zero = jnp.zeros((), jnp.float32)
    return (z, emb_weight, quantized_ste, idx_flat.reshape(b, n), loss,
            mse, mse, zero, zero, zero)
